# baseline (speedup 1.0000x reference)
import functools

import jax
import jax.numpy as jnp
from jax import lax
from jax.experimental import pallas as pl
from jax.experimental.pallas import tpu as pltpu
from jax.experimental.pallas import tpu_sc as plsc

_B = 256
_T = 1025
_NT = 1024
_D = 96
_K = 512
_OT = _K + 1
_IW = 640
_SIDE = 32.0

_NC, _NS = 2, 16
_NW = _NC * _NS
_BPW = _B // _NW


def _index_kernel(cc_ref, sd_ref, g_col_ref, g_row_ref, idx_ref):
    c0 = cc_ref[0]
    c1 = cc_ref[1]
    sd = sd_ref[0]

    def probs(shape, dim):
        t = lax.broadcasted_iota(jnp.int32, shape, dim).astype(jnp.float32)
        row = jnp.floor(t / _SIDE)
        col = t - row * _SIDE
        dist = (row - c0) * (row - c0) + (col - c1) * (col - c1)
        return 1.0 / (sd * (1.0 + dist / (sd * sd)))

    p_col = probs((_NT, 1), 0)
    p_row = probs((1, _NT), 1)
    p_sum = jnp.sum(p_row)
    s_col = jnp.log(p_col / p_sum) + g_col_ref[...]
    s_row = jnp.log(p_row / p_sum) + g_row_ref[...]

    t_col = lax.broadcasted_iota(jnp.int32, (_NT, 1), 0)
    u_row = lax.broadcasted_iota(jnp.int32, (1, _NT), 1)
    beats = (s_row > s_col) | ((s_row == s_col) & (u_row < t_col))
    rank = jnp.sum(beats.astype(jnp.float32), axis=1, keepdims=True)

    j_row = lax.broadcasted_iota(jnp.int32, (1, _IW), 1)
    sel = (rank == (j_row - 1).astype(jnp.float32))
    t_f = t_col.astype(jnp.float32)
    pos = jnp.sum(sel.astype(jnp.float32) * t_f, axis=0, keepdims=True)
    off = jnp.where(j_row == 0, 0, pos.astype(jnp.int32) + 1)

    b_col = lax.broadcasted_iota(jnp.int32, (_B, 1), 0)
    idx_ref[...] = b_col * _T + off


def _build_indices(center_coord, std_dev, g):
    g_col = g.reshape(_NT, 1)
    g_row = g.reshape(1, _NT)
    sd = jnp.asarray(std_dev, jnp.float32).reshape(1)
    return pl.pallas_call(
        _index_kernel,
        out_shape=jax.ShapeDtypeStruct((_B, _IW), jnp.int32),
        in_specs=[
            pl.BlockSpec(memory_space=pltpu.SMEM),
            pl.BlockSpec(memory_space=pltpu.SMEM),
            pl.BlockSpec(memory_space=pltpu.VMEM),
            pl.BlockSpec(memory_space=pltpu.VMEM),
        ],
        out_specs=pl.BlockSpec(memory_space=pltpu.VMEM),
    )(center_coord, sd, g_col, g_row)


def _fire_gathers(x_hbm, idx_v, rows_v, sem):
    cps = []
    for c in range(4):
        cps.append(pltpu.async_copy(
            x_hbm.at[idx_v.at[pl.ds(c * 128, 128)]],
            rows_v.at[pl.ds(c * 128, 128)], sem))
    cps.append(pltpu.async_copy(
        x_hbm.at[idx_v.at[pl.ds(512, 1)]],
        rows_v.at[pl.ds(512, 1)], sem))
    return cps


def _gather_kernel(x_hbm, idx_hbm, out_hbm, idx_v, rows_v, gsem, osem):
    wid = lax.axis_index("s") * _NC + lax.axis_index("c")
    b0 = wid * _BPW
    pltpu.sync_copy(idx_hbm.at[pl.ds(b0 * _IW, _IW)], idx_v.at[0])
    pend = _fire_gathers(x_hbm, idx_v.at[0], rows_v.at[0], gsem)
    out_cp = None
    for i in range(_BPW):
        b = b0 + i
        k = i % 2
        nk = (i + 1) % 2
        if i + 1 < _BPW:
            pltpu.sync_copy(idx_hbm.at[pl.ds((b + 1) * _IW, _IW)], idx_v.at[nk])
        for cp in pend:
            cp.wait()
        if i + 1 < _BPW:
            pend = _fire_gathers(x_hbm, idx_v.at[nk], rows_v.at[nk], gsem)
        if out_cp is not None:
            out_cp.wait()
        out_cp = pltpu.async_copy(rows_v.at[k], out_hbm.at[b], osem)
    out_cp.wait()


def _gather(x_flat, idx):
    mesh = plsc.VectorSubcoreMesh(
        core_axis_name="c", subcore_axis_name="s",
        num_cores=_NC, num_subcores=_NS)
    run = functools.partial(
        pl.kernel,
        out_type=jax.ShapeDtypeStruct((_B, _OT, _D), jnp.float32),
        mesh=mesh,
        scratch_types=[
            pltpu.VMEM((2, _IW), jnp.int32),
            pltpu.VMEM((2, _OT, _D), jnp.float32),
            pltpu.SemaphoreType.DMA,
            pltpu.SemaphoreType.DMA,
        ],
        compiler_params=pltpu.CompilerParams(use_tc_tiling_on_sc=False),
    )(_gather_kernel)
    return run(x_flat, idx.reshape(_B * _IW))


def kernel(x, center_coord, std_dev):
    g = jax.random.gumbel(jax.random.key(42), (_NT,), dtype=jnp.float32)
    idx = _build_indices(center_coord, std_dev, g)
    return _gather(x.reshape(_B * _T, _D), idx)


# speedup vs baseline: 1.0071x; 1.0071x over previous
import functools

import jax
import jax.numpy as jnp
from jax import lax
from jax.experimental import pallas as pl
from jax.experimental.pallas import tpu as pltpu
from jax.experimental.pallas import tpu_sc as plsc

_B = 256
_T = 1025
_NT = 1024
_D = 96
_K = 512
_OT = _K + 1
_IW = 640
_SIDE = 32.0

_NC, _NS = 2, 16
_NW = _NC * _NS
_BPW = _B // _NW


def _index_kernel(cc_ref, sd_ref, g_col_ref, g_row_ref, idx_ref):
    c0 = cc_ref[0]
    c1 = cc_ref[1]
    sd = sd_ref[0]

    def probs(shape, dim):
        t = lax.broadcasted_iota(jnp.int32, shape, dim).astype(jnp.float32)
        row = jnp.floor(t / _SIDE)
        col = t - row * _SIDE
        dist = (row - c0) * (row - c0) + (col - c1) * (col - c1)
        return 1.0 / (sd * (1.0 + dist / (sd * sd)))

    p_col = probs((_NT, 1), 0)
    p_row = probs((1, _NT), 1)
    p_sum = jnp.sum(p_row)
    s_col = jnp.log(p_col / p_sum) + g_col_ref[...]
    s_row = jnp.log(p_row / p_sum) + g_row_ref[...]

    t_col = lax.broadcasted_iota(jnp.int32, (_NT, 1), 0)
    u_row = lax.broadcasted_iota(jnp.int32, (1, _NT), 1)
    beats = (s_row > s_col) | ((s_row == s_col) & (u_row < t_col))
    rank = jnp.sum(beats.astype(jnp.float32), axis=1, keepdims=True)

    j_row = lax.broadcasted_iota(jnp.int32, (1, _IW), 1)
    sel = (rank == (j_row - 1).astype(jnp.float32))
    t_f = t_col.astype(jnp.float32)
    pos = jnp.sum(sel.astype(jnp.float32) * t_f, axis=0, keepdims=True)
    off = jnp.where(j_row == 0, 0, pos.astype(jnp.int32) + 1)

    idx_ref[...] = jnp.broadcast_to(off, (8, _IW))


def _build_indices(center_coord, std_dev, g):
    g_col = g.reshape(_NT, 1)
    g_row = g.reshape(1, _NT)
    sd = jnp.asarray(std_dev, jnp.float32).reshape(1)
    return pl.pallas_call(
        _index_kernel,
        out_shape=jax.ShapeDtypeStruct((8, _IW), jnp.int32),
        in_specs=[
            pl.BlockSpec(memory_space=pltpu.SMEM),
            pl.BlockSpec(memory_space=pltpu.SMEM),
            pl.BlockSpec(memory_space=pltpu.VMEM),
            pl.BlockSpec(memory_space=pltpu.VMEM),
        ],
        out_specs=pl.BlockSpec(memory_space=pltpu.VMEM),
    )(center_coord, sd, g_col, g_row)


def _fire_gathers(x_hbm, b, idx_v, rows_v, sem):
    xb = x_hbm.at[b]
    cps = []
    for c in range(4):
        cps.append(pltpu.async_copy(
            xb.at[idx_v.at[pl.ds(c * 128, 128)]],
            rows_v.at[pl.ds(c * 128, 128)], sem))
    cps.append(pltpu.async_copy(
        xb.at[idx_v.at[pl.ds(512, 1)]],
        rows_v.at[pl.ds(512, 1)], sem))
    return cps


def _gather_kernel(x_hbm, idx_hbm, out_hbm, idx_v, rows_v, gsem, osem):
    wid = lax.axis_index("s") * _NC + lax.axis_index("c")
    b0 = wid * _BPW
    pltpu.sync_copy(idx_hbm.at[pl.ds(0, _IW)], idx_v)
    pend = _fire_gathers(x_hbm, b0, idx_v, rows_v.at[0], gsem)
    out_cp = None
    for i in range(_BPW):
        b = b0 + i
        k = i % 2
        nk = (i + 1) % 2
        if out_cp is not None:
            out_cp.wait()
        if i + 1 < _BPW:
            pend_next = _fire_gathers(x_hbm, b + 1, idx_v, rows_v.at[nk], gsem)
        for cp in pend:
            cp.wait()
        out_cp = pltpu.async_copy(rows_v.at[k], out_hbm.at[b], osem)
        if i + 1 < _BPW:
            pend = pend_next
    out_cp.wait()


def _gather(x, idx):
    mesh = plsc.VectorSubcoreMesh(
        core_axis_name="c", subcore_axis_name="s",
        num_cores=_NC, num_subcores=_NS)
    run = functools.partial(
        pl.kernel,
        out_type=jax.ShapeDtypeStruct((_B, _OT, _D), jnp.float32),
        mesh=mesh,
        scratch_types=[
            pltpu.VMEM((_IW,), jnp.int32),
            pltpu.VMEM((2, _OT, _D), jnp.float32),
            pltpu.SemaphoreType.DMA,
            pltpu.SemaphoreType.DMA,
        ],
        compiler_params=pltpu.CompilerParams(use_tc_tiling_on_sc=False),
    )(_gather_kernel)
    return run(x, idx.reshape(8 * _IW))


def kernel(x, center_coord, std_dev):
    g = jax.random.gumbel(jax.random.key(42), (_NT,), dtype=jnp.float32)
    idx = _build_indices(center_coord, std_dev, g)
    return _gather(x, idx)


# speedup vs baseline: 1.1767x; 1.1684x over previous
import functools

import jax
import jax.numpy as jnp
from jax import lax
from jax.experimental import pallas as pl
from jax.experimental.pallas import tpu as pltpu
from jax.experimental.pallas import tpu_sc as plsc

_B = 256
_T = 1025
_NT = 1024
_D = 96
_K = 512
_OT = _K + 1
_SIDE = 32.0

_CH = 176
_NCHUNK = 6
_CHS = [176, 176, 176, 176, 176, 144]
_IW = 1408

_NC, _NS = 2, 16
_NW = _NC * _NS
_BPW = _B // _NW


def _index_kernel(cc_ref, sd_ref, g_col_ref, g_row_ref, idx_ref):
    c0 = cc_ref[0]
    c1 = cc_ref[1]
    sd = sd_ref[0]

    def probs(shape, dim):
        t = lax.broadcasted_iota(jnp.int32, shape, dim).astype(jnp.float32)
        row = jnp.floor(t / _SIDE)
        col = t - row * _SIDE
        dist = (row - c0) * (row - c0) + (col - c1) * (col - c1)
        return 1.0 / (sd * (1.0 + dist / (sd * sd)))

    p_col = probs((_NT, 1), 0)
    p_row = probs((1, _NT), 1)
    p_sum = jnp.sum(p_row)
    s_col = jnp.log(p_col / p_sum) + g_col_ref[...]
    s_row = jnp.log(p_row / p_sum) + g_row_ref[...]

    t_col = lax.broadcasted_iota(jnp.int32, (_NT, 1), 0)
    u_row = lax.broadcasted_iota(jnp.int32, (1, _NT), 1)

    beats = (s_row > s_col) | ((s_row == s_col) & (u_row < t_col))
    rank_col = jnp.sum(beats.astype(jnp.float32), axis=1, keepdims=True)
    kept_col = rank_col < float(_K)
    kept_row = jnp.sum(
        ((s_col > s_row) | ((s_col == s_row) & (t_col < u_row))
         ).astype(jnp.float32), axis=0, keepdims=True) < float(_K)

    nless_col = jnp.sum(
        (kept_row & (u_row < t_col)).astype(jnp.float32),
        axis=1, keepdims=True)
    i_col = nless_col + 1.0

    slot_row = lax.broadcasted_iota(jnp.int32, (1, 640), 1).astype(jnp.float32)
    e2 = (kept_col & (i_col == slot_row)).astype(jnp.float32)
    t_f = t_col.astype(jnp.float32)
    src_arr = jnp.sum((t_f + 1.0) * e2, axis=0, keepdims=True)
    dst_arr = jnp.sum((rank_col + 1.0) * e2, axis=0, keepdims=True)

    k_row = lax.broadcasted_iota(jnp.int32, (1, 128), 1)
    starts = jnp.where(k_row <= 5, k_row * _CH,
                       jnp.where(k_row == 6, _NT, _T)).astype(jnp.float32)
    cnt = jnp.sum(
        (kept_col & ((t_f + 1.0) < starts)).astype(jnp.float32),
        axis=0, keepdims=True)
    bnd = jnp.where(starts > 0.0, cnt + 1.0, 0.0)

    row = jnp.concatenate(
        [src_arr, dst_arr, bnd], axis=1).astype(jnp.int32)
    idx_ref[...] = jnp.broadcast_to(row, (8, _IW))


def _build_indices(center_coord, std_dev, g):
    g_col = g.reshape(_NT, 1)
    g_row = g.reshape(1, _NT)
    sd = jnp.asarray(std_dev, jnp.float32).reshape(1)
    return pl.pallas_call(
        _index_kernel,
        out_shape=jax.ShapeDtypeStruct((8, _IW), jnp.int32),
        in_specs=[
            pl.BlockSpec(memory_space=pltpu.SMEM),
            pl.BlockSpec(memory_space=pltpu.SMEM),
            pl.BlockSpec(memory_space=pltpu.VMEM),
            pl.BlockSpec(memory_space=pltpu.VMEM),
        ],
        out_specs=pl.BlockSpec(memory_space=pltpu.VMEM),
    )(center_coord, sd, g_col, g_row)


def _select_rows(idx_v, src_base, lo, hi, in_ref, out_v):
    def step(i, _):
        srcs = idx_v[pl.ds(i, 16)]
        dsts = idx_v[pl.ds(i + 640, 16)]
        src = srcs[0] - src_base
        dst = dsts[0]
        for k in range(_D // 16):
            out_v[dst, pl.ds(k * 16, 16)] = in_ref[src, pl.ds(k * 16, 16)]
        return 0

    lax.fori_loop(lo, hi, step, 0)


def _gather_kernel(x_hbm, lr_hbm, idx_hbm, out_hbm,
                   idx_v, in_v, out_v, lr_v, csem, osem):
    wid = lax.axis_index("s") * _NC + lax.axis_index("c")
    b0 = wid * _BPW
    pltpu.sync_copy(idx_hbm.at[pl.ds(0, _IW)], idx_v)
    bnds = idx_v[pl.ds(1280, 16)]

    out_cp = None
    for i in range(_BPW):
        b = b0 + i
        pltpu.sync_copy(lr_hbm.at[b], lr_v)
        cp = pltpu.async_copy(
            x_hbm.at[b, pl.ds(0, _CHS[0])], in_v.at[0, pl.ds(0, _CHS[0])],
            csem)
        for c in range(_NCHUNK):
            if c + 1 < _NCHUNK:
                cp_next = pltpu.async_copy(
                    x_hbm.at[b, pl.ds((c + 1) * _CH, _CHS[c + 1])],
                    in_v.at[(c + 1) % 2, pl.ds(0, _CHS[c + 1])], csem)
            cp.wait()
            if c == 0 and out_cp is not None:
                out_cp.wait()
            _select_rows(idx_v, c * _CH, bnds[c], bnds[c + 1],
                         in_v.at[c % 2], out_v)
            if c + 1 < _NCHUNK:
                cp = cp_next
        _select_rows(idx_v, _NT, bnds[6], bnds[7], lr_v, out_v)
        out_cp = pltpu.async_copy(out_v, out_hbm.at[b], osem)
    out_cp.wait()


def _gather(x, lastrow, idx):
    mesh = plsc.VectorSubcoreMesh(
        core_axis_name="c", subcore_axis_name="s",
        num_cores=_NC, num_subcores=_NS)
    run = functools.partial(
        pl.kernel,
        out_type=jax.ShapeDtypeStruct((_B, _OT, _D), jnp.float32),
        mesh=mesh,
        scratch_types=[
            pltpu.VMEM((_IW,), jnp.int32),
            pltpu.VMEM((2, _CH, _D), jnp.float32),
            pltpu.VMEM((_OT, _D), jnp.float32),
            pltpu.VMEM((8, _D), jnp.float32),
            pltpu.SemaphoreType.DMA,
            pltpu.SemaphoreType.DMA,
        ],
        compiler_params=pltpu.CompilerParams(
            use_tc_tiling_on_sc=True, needs_layout_passes=False),
    )(_gather_kernel)
    return run(x, lastrow, idx.reshape(8 * _IW))


def kernel(x, center_coord, std_dev):
    g = jax.random.gumbel(jax.random.key(42), (_NT,), dtype=jnp.float32)
    idx = _build_indices(center_coord, std_dev, g)
    lastrow = jnp.broadcast_to(x[:, _NT:, :], (_B, 8, _D))
    return _gather(x, lastrow, idx)
